# Initial kernel scaffold; baseline (speedup 1.0000x reference)
import jax
import jax.numpy as jnp
from jax.experimental import pallas as pl


def kernel(x, edge_index, batch, W1, b1, W2, b2, W3, b3, W4, b4):
    raise NotImplementedError("write your pallas kernel here")


# speedup vs baseline: 16.9929x; 16.9929x over previous
import functools

import jax
import jax.numpy as jnp
from jax import lax
from jax.experimental import pallas as pl
from jax.experimental.pallas import tpu as pltpu
from jax.experimental.pallas import tpu_sc as plsc

N = 100000
NP = 100352
E = 1600000
EP = 1605632
G = 64
ROWS = EP // 128
SLAB = NP // 16

_MESH = plsc.VectorSubcoreMesh(core_axis_name="c", subcore_axis_name="s")
_F32 = jnp.float32


def _zero_vmem(ref, nrows):
    if len(ref.shape) == 2:
        def zf(i, carry):
            ref[i, :] = jnp.zeros((16,), _F32)
            return carry
        lax.fori_loop(0, nrows, zf, 0)
    else:
        def zf(i, carry):
            ref[pl.ds(i * 16, 16)] = jnp.zeros((16,), _F32)
            return carry
        lax.fori_loop(0, nrows, zf, 0)


@functools.partial(
    pl.kernel,
    out_type=jax.ShapeDtypeStruct((2, NP), _F32),
    mesh=_MESH,
    compiler_params=pltpu.CompilerParams(use_tc_tiling_on_sc=False),
    scratch_types=[
        pltpu.VMEM_SHARED((NP,), _F32),
        pltpu.VMEM((1024,), _F32),
        pltpu.VMEM((8, 128), jnp.int32),
        pltpu.VMEM((SLAB,), _F32),
    ],
)
def _sc_deg(dst_hbm, out_hbm, acc, onesb, dstb, stg):
    c = lax.axis_index("c")
    s = lax.axis_index("s")
    def of(i, carry):
        onesb[pl.ds(i * 16, 16)] = jnp.full((16,), 1.0, _F32)
        return carry
    lax.fori_loop(0, 64, of, 0)
    _zero_vmem(stg, SLAB // 16)
    pltpu.sync_copy(stg, acc.at[pl.ds(s * SLAB, SLAB)])
    plsc.subcore_barrier()
    r0 = (c * 16 + s) * 392
    def win(w, carry):
        pltpu.sync_copy(dst_hbm.at[pl.ds(r0 + w * 8, 8)], dstb)
        for j in range(8):
            pltpu.sync_copy(onesb.at[pl.ds(j * 128, 128)],
                            acc.at[dstb.at[j]], add=True)
        return carry
    lax.fori_loop(0, 49, win, 0)
    plsc.subcore_barrier()
    pltpu.sync_copy(acc.at[pl.ds(s * SLAB, SLAB)], stg)
    pltpu.sync_copy(stg, out_hbm.at[c, pl.ds(s * SLAB, SLAB)])


@functools.partial(
    pl.kernel,
    out_type=jax.ShapeDtypeStruct((2, NP), _F32),
    mesh=_MESH,
    compiler_params=pltpu.CompilerParams(use_tc_tiling_on_sc=False),
    scratch_types=[
        pltpu.VMEM_SHARED((NP,), _F32),
        pltpu.VMEM((1024,), jnp.int32),
        pltpu.VMEM((8, 128), jnp.int32),
        pltpu.VMEM((1024,), _F32),
        pltpu.VMEM((SLAB,), _F32),
        pltpu.SemaphoreType.DMA,
    ],
)
def _sc_seg1(vals_hbm, src_hbm, dst_hbm, out_hbm, acc, idxb, dstb, rowsb, stg, sem):
    c = lax.axis_index("c")
    s = lax.axis_index("s")
    _zero_vmem(stg, SLAB // 16)
    pltpu.sync_copy(stg, acc.at[pl.ds(s * SLAB, SLAB)])
    plsc.subcore_barrier()
    r0 = (c * 16 + s) * 392
    def win(w, carry):
        pltpu.sync_copy(src_hbm.at[pl.ds((r0 + w * 8) * 128, 1024)], idxb)
        pltpu.sync_copy(dst_hbm.at[pl.ds(r0 + w * 8, 8)], dstb)
        pltpu.async_copy(vals_hbm.at[idxb], rowsb, sem).wait()
        for j in range(8):
            pltpu.sync_copy(rowsb.at[pl.ds(j * 128, 128)],
                            acc.at[dstb.at[j]], add=True)
        return carry
    lax.fori_loop(0, 49, win, 0)
    plsc.subcore_barrier()
    pltpu.sync_copy(acc.at[pl.ds(s * SLAB, SLAB)], stg)
    pltpu.sync_copy(stg, out_hbm.at[c, pl.ds(s * SLAB, SLAB)])


@functools.partial(
    pl.kernel,
    out_type=jax.ShapeDtypeStruct((4, NP, 16), _F32),
    mesh=_MESH,
    compiler_params=pltpu.CompilerParams(use_tc_tiling_on_sc=False),
    scratch_types=[
        pltpu.VMEM_SHARED((NP, 16), _F32),
        pltpu.VMEM((1024,), jnp.int32),
        pltpu.VMEM((8, 128), jnp.int32),
        pltpu.VMEM((1024, 16), _F32),
        pltpu.SemaphoreType.DMA,
    ],
)
def _sc_seg16(sflat_hbm, src4_hbm, dst_hbm, out_hbm, acc, idxb, dstb, rowsb, sem):
    c = lax.axis_index("c")
    s = lax.axis_index("s")
    for ch in range(2):
        chunk = c * 2 + ch
        _zero_vmem(rowsb, 1024)
        for q in range(8):
            pltpu.sync_copy(rowsb.at[pl.ds(0, 784)],
                            acc.at[pl.ds(s * SLAB + q * 784, 784)])
        plsc.subcore_barrier()
        r0 = s * 784
        def win(w, carry):
            pltpu.sync_copy(src4_hbm.at[chunk, pl.ds((r0 + w * 8) * 128, 1024)], idxb)
            pltpu.sync_copy(dst_hbm.at[pl.ds(r0 + w * 8, 8)], dstb)
            pltpu.async_copy(sflat_hbm.at[idxb], rowsb, sem).wait()
            for j in range(8):
                pltpu.sync_copy(rowsb.at[pl.ds(j * 128, 128)],
                                acc.at[dstb.at[j]], add=True)
            return carry
        lax.fori_loop(0, 98, win, 0)
        plsc.subcore_barrier()
        for q in range(8):
            pltpu.sync_copy(acc.at[pl.ds(s * SLAB + q * 784, 784)],
                            rowsb.at[pl.ds(0, 784)])
            pltpu.sync_copy(rowsb.at[pl.ds(0, 784)],
                            out_hbm.at[chunk, pl.ds(s * SLAB + q * 784, 784)])
        plsc.subcore_barrier()


_BS = lambda shp, idx: pl.BlockSpec(shp, idx)
_COL = _BS((1024, 1), lambda i: (i, 0))
_M64 = _BS((1024, 64), lambda i: (i, 0))
_C16 = _BS((4, 1024, 16), lambda i: (0, i, 0))
_GRID = 98


def _tc1(d0, d1, x, dis_o, s0_o, xod_o):
    deg = d0[...] + d1[...] + 1.0
    dis = lax.rsqrt(deg)
    dis_o[...] = dis
    s0_o[...] = dis * x[...]
    xod_o[...] = x[...] / deg


_tc1_call = pl.pallas_call(
    _tc1, grid=(_GRID,),
    in_specs=[_COL] * 3,
    out_specs=[_COL] * 3,
    out_shape=[jax.ShapeDtypeStruct((NP, 1), _F32)] * 3,
)


def _tc2(t00, t01, dis, xod, W1, b1, s1c_o, h1od_o):
    p0 = dis[...] * (t00[...] + t01[...]) + xod[...]
    h1 = jnp.maximum(p0 * W1[...] + b1[...], 0.0)
    d = dis[...]
    s1c_o[...] = (d * h1).reshape(1024, 4, 16).transpose(1, 0, 2)
    h1od_o[...] = (d * d) * h1


_tc2_call = pl.pallas_call(
    _tc2, grid=(_GRID,),
    in_specs=[_COL] * 4 + [_BS((1, 64), lambda i: (0, 0))] * 2,
    out_specs=[_C16, _M64],
    out_shape=[jax.ShapeDtypeStruct((4, NP, 16), _F32),
               jax.ShapeDtypeStruct((NP, 64), _F32)],
)


def _tc3(t1c, dis, h1od, W2, b2, W3, s2c_o, god_o):
    t1 = t1c[...].transpose(1, 0, 2).reshape(1024, 64)
    d = dis[...]
    q1 = d * t1 + h1od[...]
    h2 = jnp.maximum(
        jnp.dot(q1, W2[...], preferred_element_type=_F32) + b2[...], 0.0)
    g = jnp.dot(h2, W3[...], preferred_element_type=_F32)
    s2c_o[...] = (d * g).reshape(1024, 4, 16).transpose(1, 0, 2)
    god_o[...] = (d * d) * g


_tc3_call = pl.pallas_call(
    _tc3, grid=(_GRID,),
    in_specs=[_C16, _COL, _M64,
              _BS((64, 128), lambda i: (0, 0)),
              _BS((1, 128), lambda i: (0, 0)),
              _BS((128, 64), lambda i: (0, 0))],
    out_specs=[_C16, _M64],
    out_shape=[jax.ShapeDtypeStruct((4, NP, 16), _F32),
               jax.ShapeDtypeStruct((NP, 64), _F32)],
)


def _tc4(t2c, dis, god, b3, W4, s3_o, yod_o):
    t2 = t2c[...].transpose(1, 0, 2).reshape(1024, 64)
    d = dis[...]
    q2 = d * t2 + god[...]
    h3 = jnp.maximum(q2 + b3[...], 0.0)
    y = jnp.dot(h3, W4[...], preferred_element_type=_F32)
    s3_o[...] = d * y
    yod_o[...] = (d * d) * y


_tc4_call = pl.pallas_call(
    _tc4, grid=(_GRID,),
    in_specs=[_C16, _COL, _M64,
              _BS((1, 64), lambda i: (0, 0)),
              _BS((64, 1), lambda i: (0, 0))],
    out_specs=[_COL, _COL],
    out_shape=[jax.ShapeDtypeStruct((NP, 1), _F32)] * 2,
)


def _tc5(t30, t31, dis, yod, b4, batch, out_o, macc, cacc):
    i = pl.program_id(0)
    h4 = dis[...] * (t30[...] + t31[...]) + yod[...] + b4[0, 0]
    gids = lax.broadcasted_iota(jnp.int32, (1024, G), 1)
    onehot = (batch[...] == gids).astype(_F32)
    dnum = (((0,), (0,)), ((), ()))
    contrib = lax.dot_general(onehot, h4, dimension_numbers=dnum,
                              preferred_element_type=_F32)
    cnt = lax.dot_general(onehot, jnp.ones((1024, 1), _F32),
                          dimension_numbers=dnum,
                          preferred_element_type=_F32)

    @pl.when(i == 0)
    def _():
        macc[...] = jnp.zeros_like(macc)
        cacc[...] = jnp.zeros_like(cacc)

    macc[...] += contrib
    cacc[...] += cnt

    @pl.when(i == _GRID - 1)
    def _():
        out_o[...] = macc[...] / jnp.maximum(cacc[...], 1.0)


_tc5_call = pl.pallas_call(
    _tc5, grid=(_GRID,),
    in_specs=[_COL, _COL, _COL, _COL,
              _BS((1, 1), lambda i: (0, 0)),
              _COL],
    out_specs=pl.BlockSpec((G, 1), lambda i: (0, 0)),
    out_shape=jax.ShapeDtypeStruct((G, 1), _F32),
    scratch_shapes=[pltpu.VMEM((G, 1), _F32), pltpu.VMEM((G, 1), _F32)],
)


def kernel(x, edge_index, batch, W1, b1, W2, b2, W3, b3, W4, b4):
    src = edge_index[0]
    dst = edge_index[1]
    pad_e = EP - E
    pad_n = NP - N
    src_p = jnp.concatenate([src, jnp.full((pad_e,), N, jnp.int32)])
    dst_p = jnp.concatenate([dst, jnp.full((pad_e,), N, jnp.int32)])
    dst2 = dst_p.reshape(ROWS, 128)
    src4 = src_p[None, :] + (jnp.arange(4, dtype=jnp.int32) * NP)[:, None]
    x_p = jnp.concatenate([x[:, 0], jnp.zeros((pad_n,), _F32)]).reshape(NP, 1)
    batch_p = jnp.concatenate(
        [batch, jnp.full((pad_n,), -1, jnp.int32)]).reshape(NP, 1)

    degp = _sc_deg(dst2)
    dis, s0, xod = _tc1_call(degp[0].reshape(NP, 1),
                             degp[1].reshape(NP, 1), x_p)
    t0p = _sc_seg1(s0.reshape(NP), src_p, dst2)
    s1c, h1od = _tc2_call(t0p[0].reshape(NP, 1), t0p[1].reshape(NP, 1),
                          dis, xod, W1, b1.reshape(1, 64))
    t1c = _sc_seg16(s1c.reshape(4 * NP, 16), src4, dst2)
    s2c, god = _tc3_call(t1c, dis, h1od, W2, b2.reshape(1, 128), W3)
    t2c = _sc_seg16(s2c.reshape(4 * NP, 16), src4, dst2)
    s3, yod = _tc4_call(t2c, dis, god, b3.reshape(1, 64), W4)
    t3p = _sc_seg1(s3.reshape(NP), src_p, dst2)
    return _tc5_call(t3p[0].reshape(NP, 1), t3p[1].reshape(NP, 1),
                     dis, yod, b4.reshape(1, 1), batch_p)


# speedup vs baseline: 20.8837x; 1.2290x over previous
import functools

import jax
import jax.numpy as jnp
from jax import lax
from jax.experimental import pallas as pl
from jax.experimental.pallas import tpu as pltpu
from jax.experimental.pallas import tpu_sc as plsc

N = 100000
NP = 100352
E = 1600000
EP = 1605632
G = 64
ROWS = EP // 128
SLAB = NP // 16

_MESH = plsc.VectorSubcoreMesh(core_axis_name="c", subcore_axis_name="s")
_F32 = jnp.float32


def _zero_vmem(ref, nrows):
    if len(ref.shape) == 2:
        def zf(i, carry):
            ref[i, :] = jnp.zeros((16,), _F32)
            return carry
        lax.fori_loop(0, nrows, zf, 0)
    else:
        def zf(i, carry):
            ref[pl.ds(i * 16, 16)] = jnp.zeros((16,), _F32)
            return carry
        lax.fori_loop(0, nrows, zf, 0)


@functools.partial(
    pl.kernel,
    out_type=jax.ShapeDtypeStruct((2, NP), _F32),
    mesh=_MESH,
    compiler_params=pltpu.CompilerParams(use_tc_tiling_on_sc=False),
    scratch_types=[
        pltpu.VMEM_SHARED((NP,), _F32),
        pltpu.VMEM((3584,), _F32),
        pltpu.VMEM((3584,), jnp.int32),
    ],
)
def _sc_deg(dst_hbm, out_hbm, acc, onesb, dstb):
    c = lax.axis_index("c")
    s = lax.axis_index("s")
    def zf(i, carry):
        onesb[pl.ds(i * 16, 16)] = jnp.zeros((16,), _F32)
        return carry
    lax.fori_loop(0, 224, zf, 0)
    pltpu.sync_copy(onesb, acc.at[pl.ds(s * SLAB, 3584)])
    pltpu.sync_copy(onesb.at[pl.ds(0, 2688)], acc.at[pl.ds(s * SLAB + 3584, 2688)])
    def of(i, carry):
        onesb[pl.ds(i * 16, 16)] = jnp.full((16,), 1.0, _F32)
        return carry
    lax.fori_loop(0, 224, of, 0)
    plsc.subcore_barrier()
    base = (c * 16 + s) * 50176
    def win(w, carry):
        pltpu.sync_copy(dst_hbm.at[pl.ds(base + w * 3584, 3584)], dstb)
        pltpu.sync_copy(onesb, acc.at[dstb], add=True)
        return carry
    lax.fori_loop(0, 14, win, 0)
    plsc.subcore_barrier()
    pltpu.sync_copy(acc.at[pl.ds(s * SLAB, SLAB)], out_hbm.at[c, pl.ds(s * SLAB, SLAB)])


@functools.partial(
    pl.kernel,
    out_type=jax.ShapeDtypeStruct((2, NP), _F32),
    mesh=_MESH,
    compiler_params=pltpu.CompilerParams(use_tc_tiling_on_sc=False),
    scratch_types=[
        pltpu.VMEM_SHARED((NP,), _F32),
        pltpu.VMEM((3584,), jnp.int32), pltpu.VMEM((3584,), jnp.int32),
        pltpu.VMEM((3584,), jnp.int32), pltpu.VMEM((3584,), jnp.int32),
        pltpu.VMEM((3584,), _F32), pltpu.VMEM((3584,), _F32),
        pltpu.SemaphoreType.DMA, pltpu.SemaphoreType.DMA,
    ],
)
def _sc_seg1(vals_hbm, src_hbm, dst_hbm, out_hbm, acc,
             idx0, idx1, dst0, dst1, row0, row1, sem0, sem1):
    c = lax.axis_index("c")
    s = lax.axis_index("s")
    idxb = (idx0, idx1)
    dstb = (dst0, dst1)
    rowb = (row0, row1)
    sems = (sem0, sem1)
    def zf(i, carry):
        row0[pl.ds(i * 16, 16)] = jnp.zeros((16,), _F32)
        return carry
    lax.fori_loop(0, 224, zf, 0)
    pltpu.sync_copy(row0, acc.at[pl.ds(s * SLAB, 3584)])
    pltpu.sync_copy(row0.at[pl.ds(0, 2688)], acc.at[pl.ds(s * SLAB + 3584, 2688)])
    plsc.subcore_barrier()
    base = (c * 16 + s) * 50176
    NW = 14
    pltpu.sync_copy(src_hbm.at[pl.ds(base, 3584)], idx0)
    pltpu.sync_copy(dst_hbm.at[pl.ds(base, 3584)], dst0)
    pltpu.async_copy(vals_hbm.at[idx0], row0, sem0)
    def pair(p, carry):
        for b in (0, 1):
            w = p * 2 + b
            nb = 1 - b
            @pl.when(w + 1 < NW)
            def _():
                e1 = base + (w + 1) * 3584
                pltpu.sync_copy(src_hbm.at[pl.ds(e1, 3584)], idxb[nb])
                pltpu.sync_copy(dst_hbm.at[pl.ds(e1, 3584)], dstb[nb])
                pltpu.async_copy(vals_hbm.at[idxb[nb]], rowb[nb], sems[nb])
            pltpu.make_async_copy(vals_hbm.at[idxb[b]], rowb[b], sems[b]).wait()
            pltpu.sync_copy(rowb[b], acc.at[dstb[b]], add=True)
        return carry
    lax.fori_loop(0, NW // 2, pair, 0)
    plsc.subcore_barrier()
    pltpu.sync_copy(acc.at[pl.ds(s * SLAB, SLAB)], out_hbm.at[c, pl.ds(s * SLAB, SLAB)])


@functools.partial(
    pl.kernel,
    out_type=jax.ShapeDtypeStruct((4, NP, 16), _F32),
    mesh=_MESH,
    compiler_params=pltpu.CompilerParams(use_tc_tiling_on_sc=False),
    scratch_types=[
        pltpu.VMEM_SHARED((NP, 16), _F32),
        pltpu.VMEM((512,), jnp.int32), pltpu.VMEM((512,), jnp.int32),
        pltpu.VMEM((512,), jnp.int32), pltpu.VMEM((512,), jnp.int32),
        pltpu.VMEM((512, 16), _F32), pltpu.VMEM((512, 16), _F32),
        pltpu.SemaphoreType.DMA, pltpu.SemaphoreType.DMA,
    ],
)
def _sc_seg16(sflat_hbm, src4_hbm, dst_hbm, out_hbm, acc,
              idx0, idx1, dst0, dst1, row0, row1, sem0, sem1):
    c = lax.axis_index("c")
    s = lax.axis_index("s")
    idxb = (idx0, idx1)
    dstb = (dst0, dst1)
    rowb = (row0, row1)
    sems = (sem0, sem1)
    base = s * 100352
    NW = 196
    for ch in range(2):
        chunk = c * 2 + ch
        def zf(i, carry):
            row0[i, :] = jnp.zeros((16,), _F32)
            return carry
        lax.fori_loop(0, 512, zf, 0)
        for q in range(12):
            pltpu.sync_copy(row0, acc.at[pl.ds(s * SLAB + q * 512, 512)])
        pltpu.sync_copy(row0.at[pl.ds(0, 128)],
                        acc.at[pl.ds(s * SLAB + 6144, 128)])
        plsc.subcore_barrier()
        pltpu.sync_copy(src4_hbm.at[chunk, pl.ds(base, 512)], idx0)
        pltpu.sync_copy(dst_hbm.at[pl.ds(base, 512)], dst0)
        pltpu.async_copy(sflat_hbm.at[idx0], row0, sem0)
        def pair(p, carry):
            for b in (0, 1):
                w = p * 2 + b
                nb = 1 - b
                @pl.when(w + 1 < NW)
                def _():
                    e1 = base + (w + 1) * 512
                    pltpu.sync_copy(src4_hbm.at[chunk, pl.ds(e1, 512)], idxb[nb])
                    pltpu.sync_copy(dst_hbm.at[pl.ds(e1, 512)], dstb[nb])
                    pltpu.async_copy(sflat_hbm.at[idxb[nb]], rowb[nb], sems[nb])
                pltpu.make_async_copy(sflat_hbm.at[idxb[b]], rowb[b], sems[b]).wait()
                pltpu.sync_copy(rowb[b], acc.at[dstb[b]], add=True)
            return carry
        lax.fori_loop(0, NW // 2, pair, 0)
        plsc.subcore_barrier()
        pltpu.sync_copy(acc.at[pl.ds(s * SLAB, SLAB)],
                        out_hbm.at[chunk, pl.ds(s * SLAB, SLAB)])
        plsc.subcore_barrier()


_BS = lambda shp, idx: pl.BlockSpec(shp, idx)
_COL = _BS((1024, 1), lambda i: (i, 0))
_M64 = _BS((1024, 64), lambda i: (i, 0))
_C16 = _BS((4, 1024, 16), lambda i: (0, i, 0))
_GRID = 98


def _tc1(d0, d1, x, dis_o, s0_o, xod_o):
    deg = d0[...] + d1[...] + 1.0
    dis = lax.rsqrt(deg)
    dis_o[...] = dis
    s0_o[...] = dis * x[...]
    xod_o[...] = x[...] / deg


_tc1_call = pl.pallas_call(
    _tc1, grid=(_GRID,),
    in_specs=[_COL] * 3,
    out_specs=[_COL] * 3,
    out_shape=[jax.ShapeDtypeStruct((NP, 1), _F32)] * 3,
)


def _tc2(t00, t01, dis, xod, W1, b1, s1c_o, h1od_o):
    p0 = dis[...] * (t00[...] + t01[...]) + xod[...]
    h1 = jnp.maximum(p0 * W1[...] + b1[...], 0.0)
    d = dis[...]
    s1c_o[...] = (d * h1).reshape(1024, 4, 16).transpose(1, 0, 2)
    h1od_o[...] = (d * d) * h1


_tc2_call = pl.pallas_call(
    _tc2, grid=(_GRID,),
    in_specs=[_COL] * 4 + [_BS((1, 64), lambda i: (0, 0))] * 2,
    out_specs=[_C16, _M64],
    out_shape=[jax.ShapeDtypeStruct((4, NP, 16), _F32),
               jax.ShapeDtypeStruct((NP, 64), _F32)],
)


def _tc3(t1c, dis, h1od, W2, b2, W3, s2c_o, god_o):
    t1 = t1c[...].transpose(1, 0, 2).reshape(1024, 64)
    d = dis[...]
    q1 = d * t1 + h1od[...]
    h2 = jnp.maximum(
        jnp.dot(q1, W2[...], preferred_element_type=_F32) + b2[...], 0.0)
    g = jnp.dot(h2, W3[...], preferred_element_type=_F32)
    s2c_o[...] = (d * g).reshape(1024, 4, 16).transpose(1, 0, 2)
    god_o[...] = (d * d) * g


_tc3_call = pl.pallas_call(
    _tc3, grid=(_GRID,),
    in_specs=[_C16, _COL, _M64,
              _BS((64, 128), lambda i: (0, 0)),
              _BS((1, 128), lambda i: (0, 0)),
              _BS((128, 64), lambda i: (0, 0))],
    out_specs=[_C16, _M64],
    out_shape=[jax.ShapeDtypeStruct((4, NP, 16), _F32),
               jax.ShapeDtypeStruct((NP, 64), _F32)],
)


def _tc4(t2c, dis, god, b3, W4, s3_o, yod_o):
    t2 = t2c[...].transpose(1, 0, 2).reshape(1024, 64)
    d = dis[...]
    q2 = d * t2 + god[...]
    h3 = jnp.maximum(q2 + b3[...], 0.0)
    y = jnp.dot(h3, W4[...], preferred_element_type=_F32)
    s3_o[...] = d * y
    yod_o[...] = (d * d) * y


_tc4_call = pl.pallas_call(
    _tc4, grid=(_GRID,),
    in_specs=[_C16, _COL, _M64,
              _BS((1, 64), lambda i: (0, 0)),
              _BS((64, 1), lambda i: (0, 0))],
    out_specs=[_COL, _COL],
    out_shape=[jax.ShapeDtypeStruct((NP, 1), _F32)] * 2,
)


def _tc5(t30, t31, dis, yod, b4, batch, out_o, macc, cacc):
    i = pl.program_id(0)
    h4 = dis[...] * (t30[...] + t31[...]) + yod[...] + b4[0, 0]
    gids = lax.broadcasted_iota(jnp.int32, (1024, G), 1)
    onehot = (batch[...] == gids).astype(_F32)
    dnum = (((0,), (0,)), ((), ()))
    contrib = lax.dot_general(onehot, h4, dimension_numbers=dnum,
                              preferred_element_type=_F32)
    cnt = lax.dot_general(onehot, jnp.ones((1024, 1), _F32),
                          dimension_numbers=dnum,
                          preferred_element_type=_F32)

    @pl.when(i == 0)
    def _():
        macc[...] = jnp.zeros_like(macc)
        cacc[...] = jnp.zeros_like(cacc)

    macc[...] += contrib
    cacc[...] += cnt

    @pl.when(i == _GRID - 1)
    def _():
        out_o[...] = macc[...] / jnp.maximum(cacc[...], 1.0)


_tc5_call = pl.pallas_call(
    _tc5, grid=(_GRID,),
    in_specs=[_COL, _COL, _COL, _COL,
              _BS((1, 1), lambda i: (0, 0)),
              _COL],
    out_specs=pl.BlockSpec((G, 1), lambda i: (0, 0)),
    out_shape=jax.ShapeDtypeStruct((G, 1), _F32),
    scratch_shapes=[pltpu.VMEM((G, 1), _F32), pltpu.VMEM((G, 1), _F32)],
)


def kernel(x, edge_index, batch, W1, b1, W2, b2, W3, b3, W4, b4):
    src = edge_index[0]
    dst = edge_index[1]
    pad_e = EP - E
    pad_n = NP - N
    src_p = jnp.concatenate([src, jnp.full((pad_e,), N, jnp.int32)])
    dst_p = jnp.concatenate([dst, jnp.full((pad_e,), N, jnp.int32)])
    src4 = src_p[None, :] + (jnp.arange(4, dtype=jnp.int32) * NP)[:, None]
    x_p = jnp.concatenate([x[:, 0], jnp.zeros((pad_n,), _F32)]).reshape(NP, 1)
    batch_p = jnp.concatenate(
        [batch, jnp.full((pad_n,), -1, jnp.int32)]).reshape(NP, 1)

    degp = _sc_deg(dst_p)
    dis, s0, xod = _tc1_call(degp[0].reshape(NP, 1),
                             degp[1].reshape(NP, 1), x_p)
    t0p = _sc_seg1(s0.reshape(NP), src_p, dst_p)
    s1c, h1od = _tc2_call(t0p[0].reshape(NP, 1), t0p[1].reshape(NP, 1),
                          dis, xod, W1, b1.reshape(1, 64))
    t1c = _sc_seg16(s1c.reshape(4 * NP, 16), src4, dst_p)
    s2c, god = _tc3_call(t1c, dis, h1od, W2, b2.reshape(1, 128), W3)
    t2c = _sc_seg16(s2c.reshape(4 * NP, 16), src4, dst_p)
    s3, yod = _tc4_call(t2c, dis, god, b3.reshape(1, 64), W4)
    t3p = _sc_seg1(s3.reshape(NP), src_p, dst_p)
    return _tc5_call(t3p[0].reshape(NP, 1), t3p[1].reshape(NP, 1),
                     dis, yod, b4.reshape(1, 1), batch_p)


# speedup vs baseline: 23.4118x; 1.1211x over previous
import functools

import jax
import jax.numpy as jnp
from jax import lax
from jax.experimental import pallas as pl
from jax.experimental.pallas import tpu as pltpu
from jax.experimental.pallas import tpu_sc as plsc

N = 100000
NP = 100352
E = 1600000
EP = 1605632
G = 64
ROWS = EP // 128
SLAB = NP // 16

_MESH = plsc.VectorSubcoreMesh(core_axis_name="c", subcore_axis_name="s")
_F32 = jnp.float32


def _zero_vmem(ref, nrows):
    if len(ref.shape) == 2:
        def zf(i, carry):
            ref[i, :] = jnp.zeros((16,), _F32)
            return carry
        lax.fori_loop(0, nrows, zf, 0)
    else:
        def zf(i, carry):
            ref[pl.ds(i * 16, 16)] = jnp.zeros((16,), _F32)
            return carry
        lax.fori_loop(0, nrows, zf, 0)


@functools.partial(
    pl.kernel,
    out_type=jax.ShapeDtypeStruct((2, NP), _F32),
    mesh=_MESH,
    compiler_params=pltpu.CompilerParams(use_tc_tiling_on_sc=False),
    scratch_types=[
        pltpu.VMEM_SHARED((NP,), _F32),
        pltpu.VMEM((3584,), _F32),
        pltpu.VMEM((3584,), jnp.int32),
    ],
)
def _sc_deg(dst_hbm, out_hbm, acc, onesb, dstb):
    c = lax.axis_index("c")
    s = lax.axis_index("s")
    def zf(i, carry):
        onesb[pl.ds(i * 16, 16)] = jnp.zeros((16,), _F32)
        return carry
    lax.fori_loop(0, 224, zf, 0)
    pltpu.sync_copy(onesb, acc.at[pl.ds(s * SLAB, 3584)])
    pltpu.sync_copy(onesb.at[pl.ds(0, 2688)], acc.at[pl.ds(s * SLAB + 3584, 2688)])
    def of(i, carry):
        onesb[pl.ds(i * 16, 16)] = jnp.full((16,), 1.0, _F32)
        return carry
    lax.fori_loop(0, 224, of, 0)
    plsc.subcore_barrier()
    base = (c * 16 + s) * 50176
    def win(w, carry):
        pltpu.sync_copy(dst_hbm.at[pl.ds(base + w * 3584, 3584)], dstb)
        pltpu.sync_copy(onesb, acc.at[dstb], add=True)
        return carry
    lax.fori_loop(0, 14, win, 0)
    plsc.subcore_barrier()
    pltpu.sync_copy(acc.at[pl.ds(s * SLAB, SLAB)], out_hbm.at[c, pl.ds(s * SLAB, SLAB)])


@functools.partial(
    pl.kernel,
    out_type=jax.ShapeDtypeStruct((2, NP), _F32),
    mesh=_MESH,
    compiler_params=pltpu.CompilerParams(use_tc_tiling_on_sc=False),
    scratch_types=[
        pltpu.VMEM_SHARED((NP,), _F32),
        pltpu.VMEM((3584,), jnp.int32), pltpu.VMEM((3584,), jnp.int32),
        pltpu.VMEM((3584,), jnp.int32), pltpu.VMEM((3584,), jnp.int32),
        pltpu.VMEM((3584,), _F32), pltpu.VMEM((3584,), _F32),
        pltpu.SemaphoreType.DMA, pltpu.SemaphoreType.DMA,
    ],
)
def _sc_seg1(vals_hbm, src_hbm, dst_hbm, out_hbm, acc,
             idx0, idx1, dst0, dst1, row0, row1, sem0, sem1):
    c = lax.axis_index("c")
    s = lax.axis_index("s")
    idxb = (idx0, idx1)
    dstb = (dst0, dst1)
    rowb = (row0, row1)
    sems = (sem0, sem1)
    def zf(i, carry):
        row0[pl.ds(i * 16, 16)] = jnp.zeros((16,), _F32)
        return carry
    lax.fori_loop(0, 224, zf, 0)
    pltpu.sync_copy(row0, acc.at[pl.ds(s * SLAB, 3584)])
    pltpu.sync_copy(row0.at[pl.ds(0, 2688)], acc.at[pl.ds(s * SLAB + 3584, 2688)])
    plsc.subcore_barrier()
    base = (c * 16 + s) * 50176
    NW = 14
    pltpu.sync_copy(src_hbm.at[pl.ds(base, 3584)], idx0)
    pltpu.sync_copy(dst_hbm.at[pl.ds(base, 3584)], dst0)
    pltpu.async_copy(vals_hbm.at[idx0], row0, sem0)
    def pair(p, carry):
        for b in (0, 1):
            w = p * 2 + b
            nb = 1 - b
            @pl.when(w + 1 < NW)
            def _():
                e1 = base + (w + 1) * 3584
                pltpu.sync_copy(src_hbm.at[pl.ds(e1, 3584)], idxb[nb])
                pltpu.sync_copy(dst_hbm.at[pl.ds(e1, 3584)], dstb[nb])
                pltpu.async_copy(vals_hbm.at[idxb[nb]], rowb[nb], sems[nb])
            pltpu.make_async_copy(vals_hbm.at[idxb[b]], rowb[b], sems[b]).wait()
            pltpu.sync_copy(rowb[b], acc.at[dstb[b]], add=True)
        return carry
    lax.fori_loop(0, NW // 2, pair, 0)
    plsc.subcore_barrier()
    pltpu.sync_copy(acc.at[pl.ds(s * SLAB, SLAB)], out_hbm.at[c, pl.ds(s * SLAB, SLAB)])


@functools.partial(
    pl.kernel,
    out_type=jax.ShapeDtypeStruct((NP, 64), _F32),
    mesh=_MESH,
    compiler_params=pltpu.CompilerParams(use_tc_tiling_on_sc=False),
    scratch_types=[
        pltpu.VMEM_SHARED((NP, 16), _F32),
        pltpu.VMEM((512,), jnp.int32), pltpu.VMEM((512,), jnp.int32),
        pltpu.VMEM((512,), jnp.int32), pltpu.VMEM((512,), jnp.int32),
        pltpu.VMEM((512, 16), _F32), pltpu.VMEM((512, 16), _F32),
        pltpu.SemaphoreType.DMA, pltpu.SemaphoreType.DMA,
    ],
)
def _sc_seg16(s_hbm, srcq_hbm, dst_hbm, out_hbm, acc,
              idx0, idx1, dst0, dst1, row0, row1, sem0, sem1):
    c = lax.axis_index("c")
    s = lax.axis_index("s")
    idxb = (idx0, idx1)
    dstb = (dst0, dst1)
    rowb = (row0, row1)
    sems = (sem0, sem1)
    base = s * 100352
    NW = 196

    def load_idx(nb, e0, chunk):
        pltpu.sync_copy(srcq_hbm.at[pl.ds(e0, 512)], idxb[nb])
        def af(i, carry):
            idxb[nb][pl.ds(i * 16, 16)] = idxb[nb][pl.ds(i * 16, 16)] + chunk
            return carry
        lax.fori_loop(0, 32, af, 0)
        pltpu.sync_copy(dst_hbm.at[pl.ds(e0, 512)], dstb[nb])

    for ch in range(2):
        chunk = c * 2 + ch
        col = chunk * 16
        def zf(i, carry):
            row0[i, :] = jnp.zeros((16,), _F32)
            return carry
        lax.fori_loop(0, 512, zf, 0)
        for q in range(12):
            pltpu.sync_copy(row0, acc.at[pl.ds(s * SLAB + q * 512, 512)])
        pltpu.sync_copy(row0.at[pl.ds(0, 128)],
                        acc.at[pl.ds(s * SLAB + 6144, 128)])
        plsc.subcore_barrier()
        load_idx(0, base, chunk)
        pltpu.async_copy(s_hbm.at[idx0], row0, sem0)
        def pair(p, carry):
            for b in (0, 1):
                w = p * 2 + b
                nb = 1 - b
                @pl.when(w + 1 < NW)
                def _():
                    load_idx(nb, base + (w + 1) * 512, chunk)
                    pltpu.async_copy(s_hbm.at[idxb[nb]], rowb[nb], sems[nb])
                pltpu.make_async_copy(s_hbm.at[idxb[b]], rowb[b], sems[b]).wait()
                pltpu.sync_copy(rowb[b], acc.at[dstb[b]], add=True)
            return carry
        lax.fori_loop(0, NW // 2, pair, 0)
        plsc.subcore_barrier()
        pltpu.sync_copy(acc.at[pl.ds(s * SLAB, SLAB)],
                        out_hbm.at[pl.ds(s * SLAB, SLAB), pl.ds(col, 16)])
        plsc.subcore_barrier()


_BS = lambda shp, idx: pl.BlockSpec(shp, idx)
_COL = _BS((1024, 1), lambda i: (i, 0))
_M64 = _BS((1024, 64), lambda i: (i, 0))
_C16 = _BS((4, 1024, 16), lambda i: (0, i, 0))
_GRID = 98


def _tc1(d0, d1, x, dis_o, s0_o, xod_o):
    deg = d0[...] + d1[...] + 1.0
    dis = lax.rsqrt(deg)
    dis_o[...] = dis
    s0_o[...] = dis * x[...]
    xod_o[...] = x[...] / deg


_tc1_call = pl.pallas_call(
    _tc1, grid=(_GRID,),
    in_specs=[_COL] * 3,
    out_specs=[_COL] * 3,
    out_shape=[jax.ShapeDtypeStruct((NP, 1), _F32)] * 3,
)


def _tc2(t00, t01, dis, xod, W1, b1, s1c_o, h1od_o):
    p0 = dis[...] * (t00[...] + t01[...]) + xod[...]
    h1 = jnp.maximum(p0 * W1[...] + b1[...], 0.0)
    d = dis[...]
    s1c_o[...] = d * h1
    h1od_o[...] = (d * d) * h1


_tc2_call = pl.pallas_call(
    _tc2, grid=(_GRID,),
    in_specs=[_COL] * 4 + [_BS((1, 64), lambda i: (0, 0))] * 2,
    out_specs=[_M64, _M64],
    out_shape=[jax.ShapeDtypeStruct((NP, 64), _F32),
               jax.ShapeDtypeStruct((NP, 64), _F32)],
)


def _tc3(t1c, dis, h1od, W2, b2, W3, s2c_o, god_o):
    t1 = t1c[...]
    d = dis[...]
    q1 = d * t1 + h1od[...]
    h2 = jnp.maximum(
        jnp.dot(q1, W2[...], preferred_element_type=_F32) + b2[...], 0.0)
    g = jnp.dot(h2, W3[...], preferred_element_type=_F32)
    s2c_o[...] = d * g
    god_o[...] = (d * d) * g


_tc3_call = pl.pallas_call(
    _tc3, grid=(_GRID,),
    in_specs=[_M64, _COL, _M64,
              _BS((64, 128), lambda i: (0, 0)),
              _BS((1, 128), lambda i: (0, 0)),
              _BS((128, 64), lambda i: (0, 0))],
    out_specs=[_M64, _M64],
    out_shape=[jax.ShapeDtypeStruct((NP, 64), _F32),
               jax.ShapeDtypeStruct((NP, 64), _F32)],
)


def _tc4(t2c, dis, god, b3, W4, s3_o, yod_o):
    t2 = t2c[...]
    d = dis[...]
    q2 = d * t2 + god[...]
    h3 = jnp.maximum(q2 + b3[...], 0.0)
    y = jnp.dot(h3, W4[...], preferred_element_type=_F32)
    s3_o[...] = d * y
    yod_o[...] = (d * d) * y


_tc4_call = pl.pallas_call(
    _tc4, grid=(_GRID,),
    in_specs=[_M64, _COL, _M64,
              _BS((1, 64), lambda i: (0, 0)),
              _BS((64, 1), lambda i: (0, 0))],
    out_specs=[_COL, _COL],
    out_shape=[jax.ShapeDtypeStruct((NP, 1), _F32)] * 2,
)


def _tc5(t30, t31, dis, yod, b4, batch, out_o, macc, cacc):
    i = pl.program_id(0)
    h4 = dis[...] * (t30[...] + t31[...]) + yod[...] + b4[0, 0]
    gids = lax.broadcasted_iota(jnp.int32, (1024, G), 1)
    onehot = (batch[...] == gids).astype(_F32)
    dnum = (((0,), (0,)), ((), ()))
    contrib = lax.dot_general(onehot, h4, dimension_numbers=dnum,
                              preferred_element_type=_F32)
    cnt = lax.dot_general(onehot, jnp.ones((1024, 1), _F32),
                          dimension_numbers=dnum,
                          preferred_element_type=_F32)

    @pl.when(i == 0)
    def _():
        macc[...] = jnp.zeros_like(macc)
        cacc[...] = jnp.zeros_like(cacc)

    macc[...] += contrib
    cacc[...] += cnt

    @pl.when(i == _GRID - 1)
    def _():
        out_o[...] = macc[...] / jnp.maximum(cacc[...], 1.0)


_tc5_call = pl.pallas_call(
    _tc5, grid=(_GRID,),
    in_specs=[_COL, _COL, _COL, _COL,
              _BS((1, 1), lambda i: (0, 0)),
              _COL],
    out_specs=pl.BlockSpec((G, 1), lambda i: (0, 0)),
    out_shape=jax.ShapeDtypeStruct((G, 1), _F32),
    scratch_shapes=[pltpu.VMEM((G, 1), _F32), pltpu.VMEM((G, 1), _F32)],
)


def kernel(x, edge_index, batch, W1, b1, W2, b2, W3, b3, W4, b4):
    src = edge_index[0]
    dst = edge_index[1]
    pad_e = EP - E
    pad_n = NP - N
    src_p = jnp.concatenate([src, jnp.full((pad_e,), N, jnp.int32)])
    srcq = src_p * 4
    dst_p = jnp.concatenate([dst, jnp.full((pad_e,), N, jnp.int32)])
    x_p = jnp.concatenate([x[:, 0], jnp.zeros((pad_n,), _F32)]).reshape(NP, 1)
    batch_p = jnp.concatenate(
        [batch, jnp.full((pad_n,), -1, jnp.int32)]).reshape(NP, 1)

    degp = _sc_deg(dst_p)
    dis, s0, xod = _tc1_call(degp[0].reshape(NP, 1),
                             degp[1].reshape(NP, 1), x_p)
    t0p = _sc_seg1(s0.reshape(NP), src_p, dst_p)
    s1c, h1od = _tc2_call(t0p[0].reshape(NP, 1), t0p[1].reshape(NP, 1),
                          dis, xod, W1, b1.reshape(1, 64))
    t1c = _sc_seg16(s1c.reshape(NP * 4, 16), srcq, dst_p)
    s2c, god = _tc3_call(t1c, dis, h1od, W2, b2.reshape(1, 128), W3)
    t2c = _sc_seg16(s2c.reshape(NP * 4, 16), srcq, dst_p)
    s3, yod = _tc4_call(t2c, dis, god, b3.reshape(1, 64), W4)
    t3p = _sc_seg1(s3.reshape(NP), src_p, dst_p)
    return _tc5_call(t3p[0].reshape(NP, 1), t3p[1].reshape(NP, 1),
                     dis, yod, b4.reshape(1, 1), batch_p)


# speedup vs baseline: 23.8966x; 1.0207x over previous
import functools

import jax
import jax.numpy as jnp
from jax import lax
from jax.experimental import pallas as pl
from jax.experimental.pallas import tpu as pltpu
from jax.experimental.pallas import tpu_sc as plsc

N = 100000
NP = 100352
E = 1600000
EP = 1605632
G = 64
ROWS = EP // 128
SLAB = NP // 16

_MESH = plsc.VectorSubcoreMesh(core_axis_name="c", subcore_axis_name="s")
_F32 = jnp.float32


def _zero_vmem(ref, nrows):
    if len(ref.shape) == 2:
        def zf(i, carry):
            ref[i, :] = jnp.zeros((16,), _F32)
            return carry
        lax.fori_loop(0, nrows, zf, 0)
    else:
        def zf(i, carry):
            ref[pl.ds(i * 16, 16)] = jnp.zeros((16,), _F32)
            return carry
        lax.fori_loop(0, nrows, zf, 0)


@functools.partial(
    pl.kernel,
    out_type=jax.ShapeDtypeStruct((2, NP), _F32),
    mesh=_MESH,
    compiler_params=pltpu.CompilerParams(use_tc_tiling_on_sc=False),
    scratch_types=[
        pltpu.VMEM_SHARED((NP,), _F32),
        pltpu.VMEM((3584,), _F32),
        pltpu.VMEM((3584,), jnp.int32),
    ],
)
def _sc_deg(dst_hbm, out_hbm, acc, onesb, dstb):
    c = lax.axis_index("c")
    s = lax.axis_index("s")
    def zf(i, carry):
        onesb[pl.ds(i * 16, 16)] = jnp.zeros((16,), _F32)
        return carry
    lax.fori_loop(0, 224, zf, 0)
    pltpu.sync_copy(onesb, acc.at[pl.ds(s * SLAB, 3584)])
    pltpu.sync_copy(onesb.at[pl.ds(0, 2688)], acc.at[pl.ds(s * SLAB + 3584, 2688)])
    def of(i, carry):
        onesb[pl.ds(i * 16, 16)] = jnp.full((16,), 1.0, _F32)
        return carry
    lax.fori_loop(0, 224, of, 0)
    plsc.subcore_barrier()
    base = (c * 16 + s) * 50176
    def win(w, carry):
        pltpu.sync_copy(dst_hbm.at[pl.ds(base + w * 3584, 3584)], dstb)
        pltpu.sync_copy(onesb, acc.at[dstb], add=True)
        return carry
    lax.fori_loop(0, 14, win, 0)
    plsc.subcore_barrier()
    pltpu.sync_copy(acc.at[pl.ds(s * SLAB, SLAB)], out_hbm.at[c, pl.ds(s * SLAB, SLAB)])


@functools.partial(
    pl.kernel,
    out_type=jax.ShapeDtypeStruct((2, NP), _F32),
    mesh=_MESH,
    compiler_params=pltpu.CompilerParams(use_tc_tiling_on_sc=False),
    scratch_types=[
        pltpu.VMEM_SHARED((NP,), _F32),
        pltpu.VMEM((3584,), jnp.int32), pltpu.VMEM((3584,), jnp.int32),
        pltpu.VMEM((3584,), jnp.int32), pltpu.VMEM((3584,), jnp.int32),
        pltpu.VMEM((3584,), _F32), pltpu.VMEM((3584,), _F32),
        pltpu.SemaphoreType.DMA, pltpu.SemaphoreType.DMA,
    ],
)
def _sc_seg1(vals_hbm, src_hbm, dst_hbm, out_hbm, acc,
             idx0, idx1, dst0, dst1, row0, row1, sem0, sem1):
    c = lax.axis_index("c")
    s = lax.axis_index("s")
    idxb = (idx0, idx1)
    dstb = (dst0, dst1)
    rowb = (row0, row1)
    sems = (sem0, sem1)
    def zf(i, carry):
        row0[pl.ds(i * 16, 16)] = jnp.zeros((16,), _F32)
        return carry
    lax.fori_loop(0, 224, zf, 0)
    pltpu.sync_copy(row0, acc.at[pl.ds(s * SLAB, 3584)])
    pltpu.sync_copy(row0.at[pl.ds(0, 2688)], acc.at[pl.ds(s * SLAB + 3584, 2688)])
    plsc.subcore_barrier()
    base = (c * 16 + s) * 50176
    NW = 14
    pltpu.sync_copy(src_hbm.at[pl.ds(base, 3584)], idx0)
    pltpu.sync_copy(dst_hbm.at[pl.ds(base, 3584)], dst0)
    pltpu.async_copy(vals_hbm.at[idx0], row0, sem0)
    def pair(p, carry):
        for b in (0, 1):
            w = p * 2 + b
            nb = 1 - b
            @pl.when(w + 1 < NW)
            def _():
                e1 = base + (w + 1) * 3584
                pltpu.sync_copy(src_hbm.at[pl.ds(e1, 3584)], idxb[nb])
                pltpu.sync_copy(dst_hbm.at[pl.ds(e1, 3584)], dstb[nb])
                pltpu.async_copy(vals_hbm.at[idxb[nb]], rowb[nb], sems[nb])
            pltpu.make_async_copy(vals_hbm.at[idxb[b]], rowb[b], sems[b]).wait()
            pltpu.sync_copy(rowb[b], acc.at[dstb[b]], add=True)
        return carry
    lax.fori_loop(0, NW // 2, pair, 0)
    plsc.subcore_barrier()
    pltpu.sync_copy(acc.at[pl.ds(s * SLAB, SLAB)], out_hbm.at[c, pl.ds(s * SLAB, SLAB)])


@functools.partial(
    pl.kernel,
    out_type=jax.ShapeDtypeStruct((NP, 64), _F32),
    mesh=_MESH,
    compiler_params=pltpu.CompilerParams(use_tc_tiling_on_sc=False),
    scratch_types=[
        pltpu.VMEM_SHARED((NP, 16), _F32),
        pltpu.VMEM((512,), jnp.int32), pltpu.VMEM((512,), jnp.int32),
        pltpu.VMEM((512,), jnp.int32), pltpu.VMEM((512,), jnp.int32),
        pltpu.VMEM((512, 16), _F32), pltpu.VMEM((512, 16), _F32),
        pltpu.SemaphoreType.DMA, pltpu.SemaphoreType.DMA,
    ],
)
def _sc_seg16(s_hbm, srcq_hbm, dst_hbm, out_hbm, acc,
              idx0, idx1, dst0, dst1, row0, row1, sem0, sem1):
    c = lax.axis_index("c")
    s = lax.axis_index("s")
    idxb = (idx0, idx1)
    dstb = (dst0, dst1)
    rowb = (row0, row1)
    sems = (sem0, sem1)
    base = s * 100352
    NW = 196

    def load_idx(nb, e0, chunk):
        pltpu.sync_copy(srcq_hbm.at[chunk, pl.ds(e0, 512)], idxb[nb])
        pltpu.sync_copy(dst_hbm.at[pl.ds(e0, 512)], dstb[nb])

    for ch in range(2):
        chunk = c * 2 + ch
        col = chunk * 16
        def zf(i, carry):
            row0[i, :] = jnp.zeros((16,), _F32)
            return carry
        lax.fori_loop(0, 512, zf, 0)
        for q in range(12):
            pltpu.sync_copy(row0, acc.at[pl.ds(s * SLAB + q * 512, 512)])
        pltpu.sync_copy(row0.at[pl.ds(0, 128)],
                        acc.at[pl.ds(s * SLAB + 6144, 128)])
        plsc.subcore_barrier()
        load_idx(0, base, chunk)
        pltpu.async_copy(s_hbm.at[idx0], row0, sem0)
        def pair(p, carry):
            for b in (0, 1):
                w = p * 2 + b
                nb = 1 - b
                @pl.when(w + 1 < NW)
                def _():
                    load_idx(nb, base + (w + 1) * 512, chunk)
                    pltpu.async_copy(s_hbm.at[idxb[nb]], rowb[nb], sems[nb])
                pltpu.make_async_copy(s_hbm.at[idxb[b]], rowb[b], sems[b]).wait()
                pltpu.sync_copy(rowb[b], acc.at[dstb[b]], add=True)
            return carry
        lax.fori_loop(0, NW // 2, pair, 0)
        plsc.subcore_barrier()
        pltpu.sync_copy(acc.at[pl.ds(s * SLAB, SLAB)],
                        out_hbm.at[pl.ds(s * SLAB, SLAB), pl.ds(col, 16)])
        plsc.subcore_barrier()


_BS = lambda shp, idx: pl.BlockSpec(shp, idx)
_COL = _BS((1024, 1), lambda i: (i, 0))
_M64 = _BS((1024, 64), lambda i: (i, 0))
_C16 = _BS((4, 1024, 16), lambda i: (0, i, 0))
_GRID = 98


def _tc1(d0, d1, x, dis_o, s0_o, xod_o):
    deg = d0[...] + d1[...] + 1.0
    dis = lax.rsqrt(deg)
    dis_o[...] = dis
    s0_o[...] = dis * x[...]
    xod_o[...] = x[...] / deg


_tc1_call = pl.pallas_call(
    _tc1, grid=(_GRID,),
    in_specs=[_COL] * 3,
    out_specs=[_COL] * 3,
    out_shape=[jax.ShapeDtypeStruct((NP, 1), _F32)] * 3,
)


def _tc2(t00, t01, dis, xod, W1, b1, s1c_o, h1od_o):
    p0 = dis[...] * (t00[...] + t01[...]) + xod[...]
    h1 = jnp.maximum(p0 * W1[...] + b1[...], 0.0)
    d = dis[...]
    s1c_o[...] = d * h1
    h1od_o[...] = (d * d) * h1


_tc2_call = pl.pallas_call(
    _tc2, grid=(_GRID,),
    in_specs=[_COL] * 4 + [_BS((1, 64), lambda i: (0, 0))] * 2,
    out_specs=[_M64, _M64],
    out_shape=[jax.ShapeDtypeStruct((NP, 64), _F32),
               jax.ShapeDtypeStruct((NP, 64), _F32)],
)


def _tc3(t1c, dis, h1od, W2, b2, W3, s2c_o, god_o):
    t1 = t1c[...]
    d = dis[...]
    q1 = d * t1 + h1od[...]
    h2 = jnp.maximum(
        jnp.dot(q1, W2[...], preferred_element_type=_F32) + b2[...], 0.0)
    g = jnp.dot(h2, W3[...], preferred_element_type=_F32)
    s2c_o[...] = d * g
    god_o[...] = (d * d) * g


_tc3_call = pl.pallas_call(
    _tc3, grid=(_GRID,),
    in_specs=[_M64, _COL, _M64,
              _BS((64, 128), lambda i: (0, 0)),
              _BS((1, 128), lambda i: (0, 0)),
              _BS((128, 64), lambda i: (0, 0))],
    out_specs=[_M64, _M64],
    out_shape=[jax.ShapeDtypeStruct((NP, 64), _F32),
               jax.ShapeDtypeStruct((NP, 64), _F32)],
)


def _tc4(t2c, dis, god, b3, W4, s3_o, yod_o):
    t2 = t2c[...]
    d = dis[...]
    q2 = d * t2 + god[...]
    h3 = jnp.maximum(q2 + b3[...], 0.0)
    y = jnp.dot(h3, W4[...], preferred_element_type=_F32)
    s3_o[...] = d * y
    yod_o[...] = (d * d) * y


_tc4_call = pl.pallas_call(
    _tc4, grid=(_GRID,),
    in_specs=[_M64, _COL, _M64,
              _BS((1, 64), lambda i: (0, 0)),
              _BS((64, 1), lambda i: (0, 0))],
    out_specs=[_COL, _COL],
    out_shape=[jax.ShapeDtypeStruct((NP, 1), _F32)] * 2,
)


def _tc5(t30, t31, dis, yod, b4, batch, out_o, macc, cacc):
    i = pl.program_id(0)
    h4 = dis[...] * (t30[...] + t31[...]) + yod[...] + b4[0, 0]
    gids = lax.broadcasted_iota(jnp.int32, (1024, G), 1)
    onehot = (batch[...] == gids).astype(_F32)
    dnum = (((0,), (0,)), ((), ()))
    contrib = lax.dot_general(onehot, h4, dimension_numbers=dnum,
                              preferred_element_type=_F32)
    cnt = lax.dot_general(onehot, jnp.ones((1024, 1), _F32),
                          dimension_numbers=dnum,
                          preferred_element_type=_F32)

    @pl.when(i == 0)
    def _():
        macc[...] = jnp.zeros_like(macc)
        cacc[...] = jnp.zeros_like(cacc)

    macc[...] += contrib
    cacc[...] += cnt

    @pl.when(i == _GRID - 1)
    def _():
        out_o[...] = macc[...] / jnp.maximum(cacc[...], 1.0)


_tc5_call = pl.pallas_call(
    _tc5, grid=(_GRID,),
    in_specs=[_COL, _COL, _COL, _COL,
              _BS((1, 1), lambda i: (0, 0)),
              _COL],
    out_specs=pl.BlockSpec((G, 1), lambda i: (0, 0)),
    out_shape=jax.ShapeDtypeStruct((G, 1), _F32),
    scratch_shapes=[pltpu.VMEM((G, 1), _F32), pltpu.VMEM((G, 1), _F32)],
)


def kernel(x, edge_index, batch, W1, b1, W2, b2, W3, b3, W4, b4):
    src = edge_index[0]
    dst = edge_index[1]
    pad_e = EP - E
    pad_n = NP - N
    src_p = jnp.concatenate([src, jnp.full((pad_e,), N, jnp.int32)])
    srcq = (src_p * 4)[None, :] + jnp.arange(4, dtype=jnp.int32)[:, None]
    dst_p = jnp.concatenate([dst, jnp.full((pad_e,), N, jnp.int32)])
    x_p = jnp.concatenate([x[:, 0], jnp.zeros((pad_n,), _F32)]).reshape(NP, 1)
    batch_p = jnp.concatenate(
        [batch, jnp.full((pad_n,), -1, jnp.int32)]).reshape(NP, 1)

    degp = _sc_deg(dst_p)
    dis, s0, xod = _tc1_call(degp[0].reshape(NP, 1),
                             degp[1].reshape(NP, 1), x_p)
    t0p = _sc_seg1(s0.reshape(NP), src_p, dst_p)
    s1c, h1od = _tc2_call(t0p[0].reshape(NP, 1), t0p[1].reshape(NP, 1),
                          dis, xod, W1, b1.reshape(1, 64))
    t1c = _sc_seg16(s1c.reshape(NP * 4, 16), srcq, dst_p)
    s2c, god = _tc3_call(t1c, dis, h1od, W2, b2.reshape(1, 128), W3)
    t2c = _sc_seg16(s2c.reshape(NP * 4, 16), srcq, dst_p)
    s3, yod = _tc4_call(t2c, dis, god, b3.reshape(1, 64), W4)
    t3p = _sc_seg1(s3.reshape(NP), src_p, dst_p)
    return _tc5_call(t3p[0].reshape(NP, 1), t3p[1].reshape(NP, 1),
                     dis, yod, b4.reshape(1, 1), batch_p)
